# baseline (speedup 1.0000x reference)
import functools

import jax
import jax.numpy as jnp
from jax import lax
from jax.experimental import pallas as pl
from jax.experimental.pallas import tpu as pltpu
from jax.experimental.pallas import tpu_sc as plsc

POINT_INFLUENCE = 0.04 * 30.0
INV_SIGMA = 1.0 / POINT_INFLUENCE

NC = 2
NS = 16
NW = NC * NS

QPAD = 10240
QSTR = QPAD
CHW = 5120
HW = 2560
NQT = HW // 128


def _sc_gather(xyzT_flat, idx_t, n, s, ci):
    assert n % 8 == 0
    max_rows = (s + 9) // 10
    nslot = 2 * max_rows

    mesh = plsc.VectorSubcoreMesh(core_axis_name="c", subcore_axis_name="s")

    @functools.partial(
        pl.kernel,
        out_type=jax.ShapeDtypeStruct((3 * s // 8, NQT * 2, 8, 128),
                                      jnp.float32),
        mesh=mesh,
        scratch_types=[
            pltpu.VMEM((n,), jnp.float32),
            pltpu.VMEM((HW,), jnp.int32),
            pltpu.VMEM((HW,), jnp.int32),
            pltpu.VMEM((NQT, 128), jnp.float32),
            pltpu.VMEM((NQT, 128), jnp.float32),
            pltpu.SemaphoreType.DMA,
            pltpu.SemaphoreType.DMA,
            pltpu.SemaphoreType.DMA,
            pltpu.SemaphoreType.DMA,
            pltpu.SemaphoreType.DMA,
        ],
        compiler_params=pltpu.CompilerParams(needs_layout_passes=False),
    )
    def sc_kernel(xyz_hbm, idx_hbm, out_hbm, plane_v, idx_v0, idx_v1,
                  val_v0, val_v1, sem_p, sem_i0, sem_i1, sem_o0, sem_o1):
        wid = lax.axis_index("s") * NC + lax.axis_index("c")
        coord = jnp.where(wid < 11, 0, jnp.where(wid < 22, 1, 2))
        base = wid - jnp.where(wid < 11, 0, jnp.where(wid < 22, 11, 22))
        tiles = jnp.where(wid < 22, 11, 10)
        r0 = base * s // tiles
        r1 = (base + 1) * s // tiles
        nh = 2 * (r1 - r0)

        sem_i = (sem_i0, sem_i1)
        sem_o = (sem_o0, sem_o1)
        idx_bufs = (idx_v0, idx_v1)
        val_bufs = (val_v0, val_v1)

        def idx_cp(h):
            j = r0 + h // 2
            ioff = j * QSTR + ci * CHW + (h % 2) * HW
            return pltpu.make_async_copy(
                idx_hbm.at[pl.ds(ioff, HW)], idx_bufs[h % 2], sem_i[h % 2])

        def out_cp(h):
            r = coord * s + (r0 + h // 2)
            qt0 = (h % 2) * NQT
            return pltpu.make_async_copy(
                val_bufs[h % 2],
                out_hbm.at[r // 8, pl.ds(qt0, NQT), r % 8, :],
                sem_o[h % 2])

        plane_cp = pltpu.make_async_copy(
            xyz_hbm.at[pl.ds(coord * n, n)], plane_v, sem_p)
        plane_cp.start()
        idx_cp(0).start()
        idx_cp(1).start()
        plane_cp.wait()

        for h in range(nslot):
            b = h % 2
            if h >= 2:
                @pl.when(h - 2 < nh)
                def _(h=h):
                    out_cp(h - 2).wait()

            @pl.when(h < nh)
            def _(h=h, b=b):
                idx_cp(h).wait()

                @plsc.parallel_loop(0, HW, 16, unroll=8)
                def _(u):
                    iv = idx_bufs[b][pl.ds(u, 16)]
                    val_bufs[b][u // 128, pl.ds(u % 128, 16)] = (
                        plsc.load_gather(plane_v, [iv]))
                out_cp(h).start()

            if h + 2 < nslot:
                @pl.when(h + 2 < nh)
                def _(h=h):
                    idx_cp(h + 2).start()

        for h in range(nslot - 2, nslot):
            @pl.when(h < nh)
            def _(h=h):
                out_cp(h).wait()

    return sc_kernel(xyzT_flat, idx_t)


def _tc_conv(gath, cent3, kpT, w45, s, ql):
    nblk = CHW // ql
    bqt = ql // 128
    st = s // 8

    def body(gath_ref, cent_ref, kp_ref, w_ref, out_ref):
        g = gath_ref[...]
        coords = [g[c * st:(c + 1) * st] for c in range(3)]
        cents = [cent_ref[c][None, :, None, :] for c in range(3)]
        rels = [coords[c] - cents[c] for c in range(3)]
        rows = []
        for k in range(15):
            dx = rels[0] - kp_ref[0, k]
            dy = rels[1] - kp_ref[1, k]
            dz = rels[2] - kp_ref[2, k]
            d2 = jnp.maximum(dx * dx + dy * dy + dz * dz, 1e-24)
            w = jnp.maximum(1.0 - (d2 * lax.rsqrt(d2)) * INV_SIGMA, 0.0)
            for c in range(3):
                rows.append(jnp.sum(w * coords[c], axis=(0, 2)))
        h = jnp.stack(rows, axis=0).reshape(45, ql)
        out_ref[...] = lax.dot_general(
            h, w_ref[...], (((0,), (0,)), ((), ())),
            preferred_element_type=jnp.float32)

    return pl.pallas_call(
        body,
        grid=(nblk,),
        in_specs=[
            pl.BlockSpec((3 * st, bqt, 8, 128), lambda i: (0, i, 0, 0)),
            pl.BlockSpec((3, bqt, 128), lambda i: (0, i, 0)),
            pl.BlockSpec(memory_space=pltpu.SMEM),
            pl.BlockSpec((45, 64), lambda i: (0, 0)),
        ],
        out_specs=pl.BlockSpec((ql, 64), lambda i: (i, 0)),
        out_shape=jax.ShapeDtypeStruct((CHW, 64), jnp.float32),
    )(gath, cent3, kpT, w45)


def _tc_bn(x0, x1, gamma1, beta1, q, v1):
    n0 = x0.shape[0]

    def body(x0_ref, x1_ref, g_ref, b_ref, o_ref):
        v0 = x0_ref[...]
        w1 = x1_ref[...]
        rid = lax.broadcasted_iota(jnp.int32, w1.shape, 0)
        m = rid < v1
        mean = (jnp.sum(v0, axis=0, keepdims=True)
                + jnp.sum(jnp.where(m, w1, 0.0), axis=0, keepdims=True)) * (1.0 / q)
        c0 = v0 - mean
        c1 = w1 - mean
        var = (jnp.sum(c0 * c0, axis=0, keepdims=True)
               + jnp.sum(jnp.where(m, c1 * c1, 0.0), axis=0, keepdims=True)) * (1.0 / q)
        scale = lax.rsqrt(var + 1e-5) * g_ref[...]
        y0 = c0 * scale + b_ref[...]
        y1 = c1 * scale + b_ref[...]
        o_ref[0, 0:n0, :] = jnp.where(y0 >= 0, y0, 0.2 * y0)
        o_ref[0, n0 : n0 + v1, :] = jnp.where(y1 >= 0, y1, 0.2 * y1)[:v1]

    return pl.pallas_call(
        body,
        out_shape=jax.ShapeDtypeStruct((1, q, 64), jnp.float32),
    )(x0, x1, gamma1, beta1)


def kernel(xyz, centors, idx, K_points, weight, gamma, beta):
    b, n, _ = xyz.shape
    num_group = centors.shape[1]
    q = b * num_group
    s = idx.shape[0] // q

    xyzT_flat = xyz.reshape(n, 3).T.reshape(-1)
    idx_t = jnp.pad(idx.reshape(q, s).T,
                    ((0, 0), (0, QPAD - q))).reshape(-1)
    centp = jnp.pad(centors.reshape(q, 3).T, ((0, 0), (0, QPAD - q)))
    cent4 = centp.reshape(3, QPAD // 128, 128)
    kpT = K_points.T
    w45 = weight[:, 3:6, :].reshape(45, 64)

    g0 = _sc_gather(xyzT_flat, idx_t, n, s, 0)
    g1 = _sc_gather(xyzT_flat, idx_t, n, s, 1)
    nqt_c = CHW // 128
    c30 = cent4[:, :nqt_c]
    c31 = cent4[:, nqt_c:]
    c0 = _tc_conv(g0, c30, kpT, w45, s, ql=1024)
    c1 = _tc_conv(g1, c31, kpT, w45, s, ql=1024)
    return _tc_bn(c0, c1, gamma.reshape(1, 64), beta.reshape(1, 64),
                  q, q - CHW)


# speedup vs baseline: 1.0873x; 1.0873x over previous
import functools

import jax
import jax.numpy as jnp
from jax import lax
from jax.experimental import pallas as pl
from jax.experimental.pallas import tpu as pltpu
from jax.experimental.pallas import tpu_sc as plsc

POINT_INFLUENCE = 0.04 * 30.0
INV_SIGMA = 1.0 / POINT_INFLUENCE

NC = 2
NS = 16
NW = NC * NS

QPAD = 10240
QSTR = QPAD
CHW = 5120
HW = 2560
NQT = HW // 128


def _sc_gather(xyzT_flat, idx_t, n, s, ci):
    assert n % 8 == 0
    max_rows = (s + 9) // 10
    nslot = 2 * max_rows

    mesh = plsc.VectorSubcoreMesh(core_axis_name="c", subcore_axis_name="s")

    @functools.partial(
        pl.kernel,
        out_type=jax.ShapeDtypeStruct((3 * s // 8, NQT * 2, 8, 128),
                                      jnp.float32),
        mesh=mesh,
        scratch_types=[
            pltpu.VMEM((n,), jnp.float32),
            pltpu.VMEM((HW,), jnp.int32),
            pltpu.VMEM((HW,), jnp.int32),
            pltpu.VMEM((NQT, 128), jnp.float32),
            pltpu.VMEM((NQT, 128), jnp.float32),
            pltpu.SemaphoreType.DMA,
            pltpu.SemaphoreType.DMA,
            pltpu.SemaphoreType.DMA,
            pltpu.SemaphoreType.DMA,
            pltpu.SemaphoreType.DMA,
        ],
        compiler_params=pltpu.CompilerParams(needs_layout_passes=False),
    )
    def sc_kernel(xyz_hbm, idx_hbm, out_hbm, plane_v, idx_v0, idx_v1,
                  val_v0, val_v1, sem_p, sem_i0, sem_i1, sem_o0, sem_o1):
        wid = lax.axis_index("s") * NC + lax.axis_index("c")
        coord = jnp.where(wid < 11, 0, jnp.where(wid < 22, 1, 2))
        base = wid - jnp.where(wid < 11, 0, jnp.where(wid < 22, 11, 22))
        tiles = jnp.where(wid < 22, 11, 10)
        r0 = base * s // tiles
        r1 = (base + 1) * s // tiles
        nh = 2 * (r1 - r0)

        sem_i = (sem_i0, sem_i1)
        sem_o = (sem_o0, sem_o1)
        idx_bufs = (idx_v0, idx_v1)
        val_bufs = (val_v0, val_v1)

        def idx_cp(h):
            j = r0 + h // 2
            ioff = j * QSTR + ci * CHW + (h % 2) * HW
            return pltpu.make_async_copy(
                idx_hbm.at[pl.ds(ioff, HW)], idx_bufs[h % 2], sem_i[h % 2])

        def out_cp(h):
            r = coord * s + (r0 + h // 2)
            qt0 = (h % 2) * NQT
            return pltpu.make_async_copy(
                val_bufs[h % 2],
                out_hbm.at[r // 8, pl.ds(qt0, NQT), r % 8, :],
                sem_o[h % 2])

        plane_cp = pltpu.make_async_copy(
            xyz_hbm.at[pl.ds(coord * n, n)], plane_v, sem_p)
        plane_cp.start()
        idx_cp(0).start()
        idx_cp(1).start()
        plane_cp.wait()

        for h in range(nslot):
            b = h % 2
            if h >= 2:
                @pl.when(h - 2 < nh)
                def _(h=h):
                    out_cp(h - 2).wait()

            @pl.when(h < nh)
            def _(h=h, b=b):
                idx_cp(h).wait()

                @plsc.parallel_loop(0, HW, 16, unroll=8)
                def _(u):
                    iv = idx_bufs[b][pl.ds(u, 16)]
                    val_bufs[b][u // 128, pl.ds(u % 128, 16)] = (
                        plsc.load_gather(plane_v, [iv]))
                out_cp(h).start()

            if h + 2 < nslot:
                @pl.when(h + 2 < nh)
                def _(h=h):
                    idx_cp(h + 2).start()

        for h in range(nslot - 2, nslot):
            @pl.when(h < nh)
            def _(h=h):
                out_cp(h).wait()

    return sc_kernel(xyzT_flat, idx_t)


def _tc_conv(gath, cent3, kpT, w45, s, ql):
    nblk = CHW // ql
    bqt = ql // 128
    st = s // 8

    def body(gath_ref, cent_ref, kp_ref, w_ref, out_ref):
        g = gath_ref[...]
        g3 = jnp.transpose(
            g.reshape(3, st, bqt, 8, 128), (0, 1, 3, 2, 4)
        ).reshape(3, s, ql)
        xg = g3[0]
        yg = g3[1]
        zg = g3[2]
        cent = cent_ref[...].reshape(3, 1, ql)
        relx = xg - cent[0]
        rely = yg - cent[1]
        relz = zg - cent[2]
        rows = []
        for k in range(15):
            dx = relx - kp_ref[0, k]
            dy = rely - kp_ref[1, k]
            dz = relz - kp_ref[2, k]
            d2 = jnp.maximum(dx * dx + dy * dy + dz * dz, 1e-24)
            w = jnp.maximum(1.0 - (d2 * lax.rsqrt(d2)) * INV_SIGMA, 0.0)
            rows.append(jnp.sum(w * xg, axis=0, keepdims=True))
            rows.append(jnp.sum(w * yg, axis=0, keepdims=True))
            rows.append(jnp.sum(w * zg, axis=0, keepdims=True))
        h = jnp.concatenate(rows, axis=0)
        out_ref[...] = lax.dot_general(
            h, w_ref[...], (((0,), (0,)), ((), ())),
            preferred_element_type=jnp.float32)

    return pl.pallas_call(
        body,
        grid=(nblk,),
        in_specs=[
            pl.BlockSpec((3 * st, bqt, 8, 128), lambda i: (0, i, 0, 0)),
            pl.BlockSpec((3, bqt, 128), lambda i: (0, i, 0)),
            pl.BlockSpec(memory_space=pltpu.SMEM),
            pl.BlockSpec((45, 64), lambda i: (0, 0)),
        ],
        out_specs=pl.BlockSpec((ql, 64), lambda i: (i, 0)),
        out_shape=jax.ShapeDtypeStruct((CHW, 64), jnp.float32),
    )(gath, cent3, kpT, w45)


def _tc_bn(x0, x1, gamma1, beta1, q, v1):
    n0 = x0.shape[0]

    def body(x0_ref, x1_ref, g_ref, b_ref, o_ref):
        v0 = x0_ref[...]
        w1 = x1_ref[...]
        rid = lax.broadcasted_iota(jnp.int32, w1.shape, 0)
        m = rid < v1
        mean = (jnp.sum(v0, axis=0, keepdims=True)
                + jnp.sum(jnp.where(m, w1, 0.0), axis=0, keepdims=True)) * (1.0 / q)
        c0 = v0 - mean
        c1 = w1 - mean
        var = (jnp.sum(c0 * c0, axis=0, keepdims=True)
               + jnp.sum(jnp.where(m, c1 * c1, 0.0), axis=0, keepdims=True)) * (1.0 / q)
        scale = lax.rsqrt(var + 1e-5) * g_ref[...]
        y0 = c0 * scale + b_ref[...]
        y1 = c1 * scale + b_ref[...]
        o_ref[0, 0:n0, :] = jnp.where(y0 >= 0, y0, 0.2 * y0)
        o_ref[0, n0 : n0 + v1, :] = jnp.where(y1 >= 0, y1, 0.2 * y1)[:v1]

    return pl.pallas_call(
        body,
        out_shape=jax.ShapeDtypeStruct((1, q, 64), jnp.float32),
    )(x0, x1, gamma1, beta1)


def kernel(xyz, centors, idx, K_points, weight, gamma, beta):
    b, n, _ = xyz.shape
    num_group = centors.shape[1]
    q = b * num_group
    s = idx.shape[0] // q

    xyzT_flat = xyz.reshape(n, 3).T.reshape(-1)
    idx_t = jnp.pad(idx.reshape(q, s).T,
                    ((0, 0), (0, QPAD - q))).reshape(-1)
    centp = jnp.pad(centors.reshape(q, 3).T, ((0, 0), (0, QPAD - q)))
    cent4 = centp.reshape(3, QPAD // 128, 128)
    kpT = K_points.T
    w45 = weight[:, 3:6, :].reshape(45, 64)

    g0 = _sc_gather(xyzT_flat, idx_t, n, s, 0)
    g1 = _sc_gather(xyzT_flat, idx_t, n, s, 1)
    nqt_c = CHW // 128
    c30 = cent4[:, :nqt_c]
    c31 = cent4[:, nqt_c:]
    c0 = _tc_conv(g0, c30, kpT, w45, s, ql=1024)
    c1 = _tc_conv(g1, c31, kpT, w45, s, ql=1024)
    return _tc_bn(c0, c1, gamma.reshape(1, 64), beta.reshape(1, 64),
                  q, q - CHW)


# speedup vs baseline: 1.1109x; 1.0217x over previous
import functools

import jax
import jax.numpy as jnp
from jax import lax
from jax.experimental import pallas as pl
from jax.experimental.pallas import tpu as pltpu
from jax.experimental.pallas import tpu_sc as plsc

POINT_INFLUENCE = 0.04 * 30.0
INV_SIGMA = 1.0 / POINT_INFLUENCE

NC = 2
NS = 16
NW = NC * NS

QPAD = 10240
QSTR = QPAD
CHW = 5120
HW = 2560
NQT = HW // 128


def _sc_gather(xyzT_flat, idx_t, n, s):
    assert n % 8 == 0
    max_rows = (s + 9) // 10
    nslot = 2 * max_rows

    mesh = plsc.VectorSubcoreMesh(core_axis_name="c", subcore_axis_name="s")

    @functools.partial(
        pl.kernel,
        out_type=jax.ShapeDtypeStruct((3 * s // 8, NQT * 2, 8, 128),
                                      jnp.float32),
        mesh=mesh,
        scratch_types=[
            pltpu.VMEM((n,), jnp.float32),
            pltpu.VMEM((HW,), jnp.int32),
            pltpu.VMEM((HW,), jnp.int32),
            pltpu.VMEM((NQT, 128), jnp.float32),
            pltpu.VMEM((NQT, 128), jnp.float32),
            pltpu.SemaphoreType.DMA,
            pltpu.SemaphoreType.DMA,
            pltpu.SemaphoreType.DMA,
            pltpu.SemaphoreType.DMA,
            pltpu.SemaphoreType.DMA,
        ],
        compiler_params=pltpu.CompilerParams(needs_layout_passes=False),
    )
    def sc_kernel(xyz_hbm, idx_hbm, out_hbm, plane_v, idx_v0, idx_v1,
                  val_v0, val_v1, sem_p, sem_i0, sem_i1, sem_o0, sem_o1):
        wid = lax.axis_index("s") * NC + lax.axis_index("c")
        coord = jnp.where(wid < 11, 0, jnp.where(wid < 22, 1, 2))
        base = wid - jnp.where(wid < 11, 0, jnp.where(wid < 22, 11, 22))
        tiles = jnp.where(wid < 22, 11, 10)
        r0 = base * s // tiles
        r1 = (base + 1) * s // tiles
        nh = 2 * (r1 - r0)

        sem_i = (sem_i0, sem_i1)
        sem_o = (sem_o0, sem_o1)
        idx_bufs = (idx_v0, idx_v1)
        val_bufs = (val_v0, val_v1)

        def idx_cp(h):
            j = r0 + h // 2
            ioff = j * CHW + (h % 2) * HW
            return pltpu.make_async_copy(
                idx_hbm.at[pl.ds(ioff, HW)], idx_bufs[h % 2], sem_i[h % 2])

        def out_cp(h):
            r = coord * s + (r0 + h // 2)
            qt0 = (h % 2) * NQT
            return pltpu.make_async_copy(
                val_bufs[h % 2],
                out_hbm.at[r // 8, pl.ds(qt0, NQT), r % 8, :],
                sem_o[h % 2])

        plane_cp = pltpu.make_async_copy(
            xyz_hbm.at[pl.ds(coord * n, n)], plane_v, sem_p)
        plane_cp.start()
        idx_cp(0).start()
        idx_cp(1).start()
        plane_cp.wait()

        for h in range(nslot):
            b = h % 2
            if h >= 2:
                @pl.when(h - 2 < nh)
                def _(h=h):
                    out_cp(h - 2).wait()

            @pl.when(h < nh)
            def _(h=h, b=b):
                idx_cp(h).wait()

                @plsc.parallel_loop(0, HW, 16, unroll=8)
                def _(u):
                    iv = idx_bufs[b][pl.ds(u, 16)]
                    val_bufs[b][u // 128, pl.ds(u % 128, 16)] = (
                        plsc.load_gather(plane_v, [iv]))
                out_cp(h).start()

            if h + 2 < nslot:
                @pl.when(h + 2 < nh)
                def _(h=h):
                    idx_cp(h + 2).start()

        for h in range(nslot - 2, nslot):
            @pl.when(h < nh)
            def _(h=h):
                out_cp(h).wait()

    return sc_kernel(xyzT_flat, idx_t)


def _tc_conv(gath, cent3, kpT, w45, s, ql):
    nblk = CHW // ql
    bqt = ql // 128
    st = s // 8

    def body(gath_ref, cent_ref, kp_ref, w_ref, out_ref):
        g = gath_ref[...]
        g3 = jnp.transpose(
            g.reshape(3, st, bqt, 8, 128), (0, 1, 3, 2, 4)
        ).reshape(3, s, ql)
        xg = g3[0]
        yg = g3[1]
        zg = g3[2]
        cent = cent_ref[...].reshape(3, 1, ql)
        relx = xg - cent[0]
        rely = yg - cent[1]
        relz = zg - cent[2]
        rows = []
        for k in range(15):
            dx = relx - kp_ref[0, k]
            dy = rely - kp_ref[1, k]
            dz = relz - kp_ref[2, k]
            d2 = jnp.maximum(dx * dx + dy * dy + dz * dz, 1e-24)
            w = jnp.maximum(1.0 - (d2 * lax.rsqrt(d2)) * INV_SIGMA, 0.0)
            rows.append(jnp.sum(w * xg, axis=0, keepdims=True))
            rows.append(jnp.sum(w * yg, axis=0, keepdims=True))
            rows.append(jnp.sum(w * zg, axis=0, keepdims=True))
        h = jnp.concatenate(rows, axis=0)
        out_ref[...] = lax.dot_general(
            h, w_ref[...], (((0,), (0,)), ((), ())),
            preferred_element_type=jnp.float32)

    return pl.pallas_call(
        body,
        grid=(nblk,),
        in_specs=[
            pl.BlockSpec((3 * st, bqt, 8, 128), lambda i: (0, i, 0, 0)),
            pl.BlockSpec((3, bqt, 128), lambda i: (0, i, 0)),
            pl.BlockSpec(memory_space=pltpu.SMEM),
            pl.BlockSpec((45, 64), lambda i: (0, 0)),
        ],
        out_specs=pl.BlockSpec((ql, 64), lambda i: (i, 0)),
        out_shape=jax.ShapeDtypeStruct((CHW, 64), jnp.float32),
    )(gath, cent3, kpT, w45)


def _tc_bn(x0, x1, gamma1, beta1, q, v1):
    n0 = x0.shape[0]

    def body(x0_ref, x1_ref, g_ref, b_ref, o_ref):
        v0 = x0_ref[...]
        w1 = x1_ref[...]
        rid = lax.broadcasted_iota(jnp.int32, w1.shape, 0)
        m = rid < v1
        mean = (jnp.sum(v0, axis=0, keepdims=True)
                + jnp.sum(jnp.where(m, w1, 0.0), axis=0, keepdims=True)) * (1.0 / q)
        c0 = v0 - mean
        c1 = w1 - mean
        var = (jnp.sum(c0 * c0, axis=0, keepdims=True)
               + jnp.sum(jnp.where(m, c1 * c1, 0.0), axis=0, keepdims=True)) * (1.0 / q)
        scale = lax.rsqrt(var + 1e-5) * g_ref[...]
        y0 = c0 * scale + b_ref[...]
        y1 = c1 * scale + b_ref[...]
        o_ref[0, 0:n0, :] = jnp.where(y0 >= 0, y0, 0.2 * y0)
        o_ref[0, n0 : n0 + v1, :] = jnp.where(y1 >= 0, y1, 0.2 * y1)[:v1]

    return pl.pallas_call(
        body,
        out_shape=jax.ShapeDtypeStruct((1, q, 64), jnp.float32),
    )(x0, x1, gamma1, beta1)


def kernel(xyz, centors, idx, K_points, weight, gamma, beta):
    b, n, _ = xyz.shape
    num_group = centors.shape[1]
    q = b * num_group
    s = idx.shape[0] // q

    xyzT_flat = xyz.reshape(n, 3).T.reshape(-1)
    idx2 = idx.reshape(q, s)
    idx_t0 = idx2[:CHW].T.reshape(-1)
    idx_t1 = jnp.pad(idx2[CHW:].T,
                     ((0, 0), (0, QPAD - q))).reshape(-1)
    centp = jnp.pad(centors.reshape(q, 3).T, ((0, 0), (0, QPAD - q)))
    cent4 = centp.reshape(3, QPAD // 128, 128)
    kpT = K_points.T
    w45 = weight[:, 3:6, :].reshape(45, 64)

    g0 = _sc_gather(xyzT_flat, idx_t0, n, s)
    g1 = _sc_gather(xyzT_flat, idx_t1, n, s)
    nqt_c = CHW // 128
    c30 = cent4[:, :nqt_c]
    c31 = cent4[:, nqt_c:]
    c0 = _tc_conv(g0, c30, kpT, w45, s, ql=1024)
    c1 = _tc_conv(g1, c31, kpT, w45, s, ql=1024)
    return _tc_bn(c0, c1, gamma.reshape(1, 64), beta.reshape(1, 64),
                  q, q - CHW)
